# baseline (speedup 1.0000x reference)
import functools

import jax
import jax.numpy as jnp
from jax import lax
from jax.experimental import pallas as pl
from jax.experimental.pallas import tpu as pltpu
from jax.experimental.pallas import tpu_sc as plsc

N = 100000
E = 6400000
NPAD = 100352
NHALF = 50176
DUMP = 50008
NC = 2
NS = 16
C1 = 800
C2 = 640
B2 = 512
SEL = 1168
L1_CHUNKS = E // NC // NS // C1
L2_CHUNKS = E // NS // C2

def _pieces(total, c):
    out, r = [], 0
    while r < total:
        out.append(min(c, total - r)); r += out[-1]
    return out

_mesh = plsc.VectorSubcoreMesh(core_axis_name="c", subcore_axis_name="s",
                               num_cores=NC, num_subcores=NS)
_sc_params = pltpu.CompilerParams(needs_layout_passes=False,
                                  use_tc_tiling_on_sc=False)


def _tc_sigmoid_body(em_ref, sig_ref):
    m = em_ref[...]
    sig_ref[...] = (1.0 / (1.0 + jnp.exp(-m))).reshape(-1)


def _tc_sigmoid(edge_mask):
    B = 32768
    return pl.pallas_call(
        _tc_sigmoid_body,
        grid=(pl.cdiv(E // 2, B),),
        in_specs=[pl.BlockSpec((B, 1), lambda i: (i, 0))],
        out_specs=pl.BlockSpec((B,), lambda i: (i,)),
        out_shape=jax.ShapeDtypeStruct((E // 2,), jnp.float32),
    )(edge_mask)


def _zero_rows(rows_v, nrows, ncols):
    z = jnp.zeros((16,), jnp.float32)
    def body(i, _):
        for c0 in range(0, ncols, 16):
            rows_v[i, pl.ds(c0, 16)] = z
        return 0
    lax.fori_loop(0, nrows, body, 0)


@functools.partial(
    pl.kernel,
    out_type=jax.ShapeDtypeStruct((NC, NPAD, 16), jnp.float32),
    mesh=_mesh,
    scratch_types=[
        pltpu.VMEM_SHARED((NPAD, 16), jnp.float32),
        pltpu.VMEM((C1,), jnp.int32),
        pltpu.VMEM((C1,), jnp.int32),
        pltpu.VMEM((C1 // 2,), jnp.float32),
        pltpu.VMEM((C1, 4), jnp.float32),
        pltpu.VMEM((C1, 16), jnp.float32),
        pltpu.SemaphoreType.DMA,
    ],
    compiler_params=_sc_params,
)
def _sc_layer1(ei_hbm, sg_hbm, ef_hbm, xpad_hbm, out_hbm,
               acc_sh, src_v, dst_v, sig_v, ef_v, rows_v, sem):
    cid = lax.axis_index("c")
    sid = lax.axis_index("s")
    iota = lax.iota(jnp.int32, 16)
    ones = jnp.ones((16,), jnp.float32)
    f15 = jnp.full((16,), 15, jnp.int32)

    _zero_rows(rows_v, C1, 16)
    rows_per_tile = NPAD // NS
    r0 = sid * rows_per_tile
    for p in _pieces(rows_per_tile, C1):
        pltpu.sync_copy(rows_v.at[pl.ds(0, p)], acc_sh.at[pl.ds(r0, p)])
        r0 = r0 + p
    plsc.subcore_barrier()

    tile_base = cid * (E // NC) + sid * (E // NC // NS)

    def chunk(ch, _):
        eb = pl.multiple_of(tile_base + ch * C1, 8)
        mb = pl.multiple_of((tile_base + ch * C1) // 2, 8)
        pltpu.sync_copy(ei_hbm.at[0, pl.ds(eb, C1)], src_v)
        pltpu.sync_copy(ei_hbm.at[1, pl.ds(eb, C1)], dst_v)
        pltpu.sync_copy(sg_hbm.at[pl.ds(mb, C1 // 2)], sig_v)
        pltpu.sync_copy(ef_hbm.at[pl.ds(eb, C1), :], ef_v)
        pltpu.async_copy(xpad_hbm.at[src_v], rows_v, sem).wait()

        def group(g, _):
            eids = g * 16 + iota
            maskv = plsc.load_gather(sig_v, [lax.shift_right_logical(eids, 1)])
            for f in range(4):
                fv = jnp.full((16,), f, jnp.int32)
                v = plsc.load_gather(ef_v, [eids, fv]) * maskv
                plsc.store_scatter(rows_v, [eids, fv], v)
            for f in range(4, 11):
                fv = jnp.full((16,), f, jnp.int32)
                v = plsc.load_gather(rows_v, [eids, fv]) * maskv
                plsc.store_scatter(rows_v, [eids, fv], v)
            plsc.store_scatter(rows_v, [eids, f15], ones)
            return 0
        lax.fori_loop(0, C1 // 16, group, 0)
        pltpu.sync_copy(rows_v, acc_sh.at[dst_v], add=True)
        return 0

    lax.fori_loop(0, L1_CHUNKS, chunk, 0)
    plsc.subcore_barrier()

    r0 = sid * rows_per_tile
    for p in _pieces(rows_per_tile, C1):
        pltpu.sync_copy(acc_sh.at[pl.ds(r0, p)], rows_v.at[pl.ds(0, p)])
        pltpu.sync_copy(rows_v.at[pl.ds(0, p)], out_hbm.at[cid].at[pl.ds(r0, p)])
        r0 = r0 + p


@functools.partial(
    pl.kernel,
    out_type=jax.ShapeDtypeStruct((NC, NHALF, 32), jnp.float32),
    mesh=_mesh,
    scratch_types=[
        pltpu.VMEM_SHARED((NHALF, 32), jnp.float32),
        pltpu.VMEM((C2,), jnp.int32),
        pltpu.VMEM((C2,), jnp.int32),
        pltpu.VMEM((C2 // 2,), jnp.float32),
        pltpu.VMEM((SEL,), jnp.int32),
        pltpu.VMEM((SEL,), jnp.int32),
        pltpu.VMEM((SEL,), jnp.float32),
        pltpu.VMEM((B2,), jnp.int32),
        pltpu.VMEM((B2, 32), jnp.float32),
        pltpu.SemaphoreType.DMA,
    ],
    compiler_params=_sc_params,
)
def _sc_layer2(ei_hbm, sg_hbm, h_hbm, out_hbm,
               acc_sh, src_v, dst_v, sig_v, ssel_v, dsel_v, msel_v,
               dbat_v, rows_v, sem):
    cid = lax.axis_index("c")
    sid = lax.axis_index("s")
    iota = lax.iota(jnp.int32, 16)
    lo = cid * 50000

    _zero_rows(rows_v, B2, 32)
    rows_per_tile = NHALF // NS
    r0 = sid * rows_per_tile
    for p in _pieces(rows_per_tile, B2):
        pltpu.sync_copy(rows_v.at[pl.ds(0, p)], acc_sh.at[pl.ds(r0, p)])
        r0 = r0 + p
    plsc.subcore_barrier()

    def process_batch():
        for j in range(B2 // 16):
            dbat_v[pl.ds(j * 16, 16)] = dsel_v[pl.ds(j * 16, 16)]
        pltpu.async_copy(h_hbm.at[ssel_v.at[pl.ds(0, B2)]], rows_v, sem).wait()

        def scale(g, _):
            eids = g * 16 + iota
            maskv = msel_v[pl.ds(g * 16, 16)]
            for f in range(32):
                fv = jnp.full((16,), f, jnp.int32)
                v = plsc.load_gather(rows_v, [eids, fv]) * maskv
                plsc.store_scatter(rows_v, [eids, fv], v)
            return 0
        lax.fori_loop(0, B2 // 16, scale, 0)
        pltpu.sync_copy(rows_v, acc_sh.at[dbat_v], add=True)
        for j in range((SEL - B2) // 16):
            ssel_v[pl.ds(j * 16, 16)] = ssel_v[pl.ds(B2 + j * 16, 16)]
            dsel_v[pl.ds(j * 16, 16)] = dsel_v[pl.ds(B2 + j * 16, 16)]
            msel_v[pl.ds(j * 16, 16)] = msel_v[pl.ds(B2 + j * 16, 16)]

    tile_base = sid * (E // NS)

    def chunk(ch, cnt_v):
        eb = pl.multiple_of(tile_base + ch * C2, 8)
        mb = pl.multiple_of((tile_base + ch * C2) // 2, 8)
        pltpu.sync_copy(ei_hbm.at[0, pl.ds(eb, C2)], src_v)
        pltpu.sync_copy(ei_hbm.at[1, pl.ds(eb, C2)], dst_v)
        pltpu.sync_copy(sg_hbm.at[pl.ds(mb, C2 // 2)], sig_v)

        def group(g, cnt_v):
            eids = g * 16 + iota
            s16 = src_v[pl.ds(g * 16, 16)]
            d16 = dst_v[pl.ds(g * 16, 16)] - lo
            inr = jnp.logical_and(d16 >= 0, d16 < 50000)
            mv = plsc.load_gather(sig_v, [lax.shift_right_logical(eids, 1)])
            cum = plsc.cumsum(jnp.where(inr, 1, 0))
            pos = cnt_v + cum - 1
            plsc.store_scatter(ssel_v, [pos], s16, mask=inr)
            plsc.store_scatter(dsel_v, [pos], d16, mask=inr)
            plsc.store_scatter(msel_v, [pos], mv, mask=inr)
            return cnt_v + plsc.all_reduce_population_count(inr)
        cnt_v = lax.fori_loop(0, C2 // 16, group, cnt_v)

        for _rep in range(2):
            t = jnp.max(cnt_v)

            @pl.when(t >= B2)
            def _proc():
                process_batch()
            cnt_v = jnp.where(cnt_v >= B2, cnt_v - B2, cnt_v)
        return cnt_v

    cnt_v = lax.fori_loop(0, L2_CHUNKS, chunk, jnp.zeros((16,), jnp.int32))

    c_end = jnp.max(cnt_v)
    for j in range(B2 // 16):
        lm = (j * 16 + iota) >= c_end
        sl = pl.ds(j * 16, 16)
        msel_v[sl] = jnp.where(lm, 0.0, msel_v[sl])
        ssel_v[sl] = jnp.where(lm, 0, ssel_v[sl])
        dsel_v[sl] = jnp.where(lm, DUMP, dsel_v[sl])
    process_batch()
    plsc.subcore_barrier()

    r0 = sid * rows_per_tile
    for p in _pieces(rows_per_tile, B2):
        pltpu.sync_copy(acc_sh.at[pl.ds(r0, p)], rows_v.at[pl.ds(0, p)])
        pltpu.sync_copy(rows_v.at[pl.ds(0, p)], out_hbm.at[cid].at[pl.ds(r0, p)])
        r0 = r0 + p


def _tc_dense1_body(p_ref, w1a_ref, b1a_ref, w1b_ref, b1b_ref, h_ref):
    s = p_ref[0] + p_ref[1]
    deg = s[:, 15:16]
    hn = s * (1.0 / jnp.maximum(deg, 1.0))
    h1 = jnp.maximum(
        jnp.dot(hn, w1a_ref[...], preferred_element_type=jnp.float32)
        + b1a_ref[...], 0.0)
    h2 = jnp.maximum(
        jnp.dot(h1, w1b_ref[...], preferred_element_type=jnp.float32)
        + b1b_ref[...], 0.0)
    h_ref[...] = h2


def _tc_dense1(out1, w1a_pad, b1a, w1bT, b1b):
    B = 2048
    grid = (NPAD // B,)
    return pl.pallas_call(
        _tc_dense1_body,
        grid=grid,
        in_specs=[
            pl.BlockSpec((NC, B, 16), lambda i: (0, i, 0)),
            pl.BlockSpec((16, 32), lambda i: (0, 0)),
            pl.BlockSpec((1, 32), lambda i: (0, 0)),
            pl.BlockSpec((32, 32), lambda i: (0, 0)),
            pl.BlockSpec((1, 32), lambda i: (0, 0)),
        ],
        out_specs=pl.BlockSpec((B, 32), lambda i: (i, 0)),
        out_shape=jax.ShapeDtypeStruct((NPAD, 32), jnp.float32),
    )(out1, w1a_pad, b1a, w1bT, b1b)


def _tc_dense2_body(nsteps, a_ref, s1_ref, w2a_ref, b2a_ref, w2b_ref, b2b_ref,
                    wm1_ref, bm1_ref, wm2_ref, bm2_ref, out_ref, gmax):
    i = pl.program_id(0)

    @pl.when(i == 0)
    def _init():
        gmax[...] = jnp.full((1, 32), -jnp.inf, jnp.float32)

    a = a_ref[0]
    deg = s1_ref[0][:, 15:16]
    hn = a * (1.0 / jnp.maximum(deg, 1.0))
    h1 = jnp.maximum(
        jnp.dot(hn, w2a_ref[...], preferred_element_type=jnp.float32)
        + b2a_ref[...], 0.0)
    h2 = jnp.maximum(
        jnp.dot(h1, w2b_ref[...], preferred_element_type=jnp.float32)
        + b2b_ref[...], 0.0)
    gmax[...] = jnp.maximum(gmax[...], jnp.max(h2, axis=0, keepdims=True))

    @pl.when(i == nsteps - 1)
    def _head():
        g = gmax[...]
        z1 = jnp.maximum(
            jnp.dot(g, wm1_ref[...], preferred_element_type=jnp.float32)
            + bm1_ref[...], 0.0)
        logits = (jnp.dot(z1, wm2_ref[...], preferred_element_type=jnp.float32)
                  + bm2_ref[...])
        m = jnp.max(logits, axis=1, keepdims=True)
        e = jnp.exp(logits - m)
        out_ref[...] = e / jnp.sum(e, axis=1, keepdims=True)


def _tc_dense2(acc2, out1, w2aT, b2a, w2bT, b2b, wm1T, bm1, wm2T, bm2):
    B = 2000
    per_part = 50000 // B
    nsteps = NC * per_part
    return pl.pallas_call(
        functools.partial(_tc_dense2_body, nsteps),
        grid=(nsteps,),
        in_specs=[
            pl.BlockSpec((1, B, 32), lambda i: (i // 25, i % 25, 0)),
            pl.BlockSpec((1, B, 16), lambda i: (0, i, 0)),
            pl.BlockSpec((32, 32), lambda i: (0, 0)),
            pl.BlockSpec((1, 32), lambda i: (0, 0)),
            pl.BlockSpec((32, 32), lambda i: (0, 0)),
            pl.BlockSpec((1, 32), lambda i: (0, 0)),
            pl.BlockSpec((32, 16), lambda i: (0, 0)),
            pl.BlockSpec((1, 16), lambda i: (0, 0)),
            pl.BlockSpec((16, 2), lambda i: (0, 0)),
            pl.BlockSpec((1, 2), lambda i: (0, 0)),
        ],
        out_specs=pl.BlockSpec((1, 2), lambda i: (0, 0)),
        out_shape=jax.ShapeDtypeStruct((1, 2), jnp.float32),
        scratch_shapes=[pltpu.VMEM((1, 32), jnp.float32)],
    )(acc2, out1, w2aT, b2a, w2bT, b2b, wm1T, bm1, wm2T, bm2)


def kernel(x, edge_feat, edge_index, edge_mask,
           W1a, b1a, W1b, b1b, W2a, b2a, W2b, b2b, Wm1, bm1, Wm2, bm2):
    ei = edge_index.astype(jnp.int32)
    sg = _tc_sigmoid(edge_mask)

    x_pad = jnp.pad(x, ((0, NPAD - N), (4, 5)))
    w1a_pad = jnp.zeros((16, 32), jnp.float32).at[:11, :].set(W1a.T)

    out1 = _sc_layer1(ei, sg, edge_feat, x_pad)
    h = _tc_dense1(out1, w1a_pad, b1a.reshape(1, 32), W1b.T,
                   b1b.reshape(1, 32))
    acc2 = _sc_layer2(ei, sg, h)
    pred = _tc_dense2(acc2, out1, W2a.T, b2a.reshape(1, 32), W2b.T,
                      b2b.reshape(1, 32), Wm1.T, bm1.reshape(1, 16),
                      Wm2.T, bm2.reshape(1, 2))
    return pred


# speedup vs baseline: 1.0006x; 1.0006x over previous
import functools

import jax
import jax.numpy as jnp
from jax import lax
from jax.experimental import pallas as pl
from jax.experimental.pallas import tpu as pltpu
from jax.experimental.pallas import tpu_sc as plsc

N = 100000
E = 6400000
NPAD = 100352
NHALF = 50176
DUMP = 50008
NC = 2
NS = 16
C1 = 800
C2 = 640
B2 = 512
SEL = 1168
L1_CHUNKS = E // NC // NS // C1
L2_CHUNKS = E // NS // C2

def _pieces(total, c):
    out, r = [], 0
    while r < total:
        out.append(min(c, total - r)); r += out[-1]
    return out

_mesh = plsc.VectorSubcoreMesh(core_axis_name="c", subcore_axis_name="s",
                               num_cores=NC, num_subcores=NS)
_sc_params = pltpu.CompilerParams(needs_layout_passes=False,
                                  use_tc_tiling_on_sc=False)


def _tc_sigmoid_body(em_ref, sig_ref):
    m = em_ref[...]
    sig_ref[...] = (1.0 / (1.0 + jnp.exp(-m))).reshape(-1)


def _tc_sigmoid(edge_mask):
    B = 32768
    return pl.pallas_call(
        _tc_sigmoid_body,
        grid=(pl.cdiv(E // 2, B),),
        in_specs=[pl.BlockSpec((B, 1), lambda i: (i, 0))],
        out_specs=pl.BlockSpec((B,), lambda i: (i,)),
        out_shape=jax.ShapeDtypeStruct((E // 2,), jnp.float32),
    )(edge_mask)


def _tc_split_body(ei_ref, src_ref, dst_ref):
    e = ei_ref[...]
    src_ref[...] = e[0]
    dst_ref[...] = e[1]


def _tc_split(ei):
    B = 32768
    return pl.pallas_call(
        _tc_split_body,
        grid=(pl.cdiv(E, B),),
        in_specs=[pl.BlockSpec((2, B), lambda i: (0, i))],
        out_specs=[pl.BlockSpec((B,), lambda i: (i,)),
                   pl.BlockSpec((B,), lambda i: (i,))],
        out_shape=[jax.ShapeDtypeStruct((E,), jnp.int32),
                   jax.ShapeDtypeStruct((E,), jnp.int32)],
    )(ei)


def _zero_rows(rows_v, nrows, ncols):
    z = jnp.zeros((16,), jnp.float32)
    def body(i, _):
        for c0 in range(0, ncols, 16):
            rows_v[i, pl.ds(c0, 16)] = z
        return 0
    lax.fori_loop(0, nrows, body, 0)


@functools.partial(
    pl.kernel,
    out_type=jax.ShapeDtypeStruct((NC, NPAD, 16), jnp.float32),
    mesh=_mesh,
    scratch_types=[
        pltpu.VMEM_SHARED((NPAD, 16), jnp.float32),
        pltpu.VMEM((C1,), jnp.int32),
        pltpu.VMEM((C1,), jnp.int32),
        pltpu.VMEM((C1 // 2,), jnp.float32),
        pltpu.VMEM((C1, 4), jnp.float32),
        pltpu.VMEM((C1, 16), jnp.float32),
        pltpu.SemaphoreType.DMA,
    ],
    compiler_params=_sc_params,
)
def _sc_layer1(src_hbm, dst_hbm, sg_hbm, ef_hbm, xpad_hbm, out_hbm,
               acc_sh, src_v, dst_v, sig_v, ef_v, rows_v, sem):
    cid = lax.axis_index("c")
    sid = lax.axis_index("s")
    iota = lax.iota(jnp.int32, 16)
    ones = jnp.ones((16,), jnp.float32)
    f15 = jnp.full((16,), 15, jnp.int32)

    _zero_rows(rows_v, C1, 16)
    rows_per_tile = NPAD // NS
    r0 = sid * rows_per_tile
    for p in _pieces(rows_per_tile, C1):
        pltpu.sync_copy(rows_v.at[pl.ds(0, p)], acc_sh.at[pl.ds(r0, p)])
        r0 = r0 + p
    plsc.subcore_barrier()

    tile_base = cid * (E // NC) + sid * (E // NC // NS)

    def chunk(ch, _):
        eb = pl.multiple_of(tile_base + ch * C1, 8)
        mb = pl.multiple_of((tile_base + ch * C1) // 2, 8)
        pltpu.sync_copy(src_hbm.at[pl.ds(eb, C1)], src_v)
        pltpu.sync_copy(dst_hbm.at[pl.ds(eb, C1)], dst_v)
        pltpu.sync_copy(sg_hbm.at[pl.ds(mb, C1 // 2)], sig_v)
        pltpu.sync_copy(ef_hbm.at[pl.ds(eb, C1), :], ef_v)
        pltpu.async_copy(xpad_hbm.at[src_v], rows_v, sem).wait()

        def group(g, _):
            eids = g * 16 + iota
            maskv = plsc.load_gather(sig_v, [lax.shift_right_logical(eids, 1)])
            for f in range(4):
                fv = jnp.full((16,), f, jnp.int32)
                v = plsc.load_gather(ef_v, [eids, fv]) * maskv
                plsc.store_scatter(rows_v, [eids, fv], v)
            for f in range(4, 11):
                fv = jnp.full((16,), f, jnp.int32)
                v = plsc.load_gather(rows_v, [eids, fv]) * maskv
                plsc.store_scatter(rows_v, [eids, fv], v)
            plsc.store_scatter(rows_v, [eids, f15], ones)
            return 0
        lax.fori_loop(0, C1 // 16, group, 0)
        pltpu.sync_copy(rows_v, acc_sh.at[dst_v], add=True)
        return 0

    lax.fori_loop(0, L1_CHUNKS, chunk, 0)
    plsc.subcore_barrier()

    r0 = sid * rows_per_tile
    for p in _pieces(rows_per_tile, C1):
        pltpu.sync_copy(acc_sh.at[pl.ds(r0, p)], rows_v.at[pl.ds(0, p)])
        pltpu.sync_copy(rows_v.at[pl.ds(0, p)], out_hbm.at[cid].at[pl.ds(r0, p)])
        r0 = r0 + p


@functools.partial(
    pl.kernel,
    out_type=jax.ShapeDtypeStruct((NC, NHALF, 32), jnp.float32),
    mesh=_mesh,
    scratch_types=[
        pltpu.VMEM_SHARED((NHALF, 32), jnp.float32),
        pltpu.VMEM((C2,), jnp.int32),
        pltpu.VMEM((C2,), jnp.int32),
        pltpu.VMEM((C2 // 2,), jnp.float32),
        pltpu.VMEM((SEL,), jnp.int32),
        pltpu.VMEM((SEL,), jnp.int32),
        pltpu.VMEM((SEL,), jnp.float32),
        pltpu.VMEM((B2,), jnp.int32),
        pltpu.VMEM((B2, 32), jnp.float32),
        pltpu.SemaphoreType.DMA,
    ],
    compiler_params=_sc_params,
)
def _sc_layer2(src_hbm, dst_hbm, sg_hbm, h_hbm, out_hbm,
               acc_sh, src_v, dst_v, sig_v, ssel_v, dsel_v, msel_v,
               dbat_v, rows_v, sem):
    cid = lax.axis_index("c")
    sid = lax.axis_index("s")
    iota = lax.iota(jnp.int32, 16)
    lo = cid * 50000

    _zero_rows(rows_v, B2, 32)
    rows_per_tile = NHALF // NS
    r0 = sid * rows_per_tile
    for p in _pieces(rows_per_tile, B2):
        pltpu.sync_copy(rows_v.at[pl.ds(0, p)], acc_sh.at[pl.ds(r0, p)])
        r0 = r0 + p
    plsc.subcore_barrier()

    def process_batch():
        for j in range(B2 // 16):
            dbat_v[pl.ds(j * 16, 16)] = dsel_v[pl.ds(j * 16, 16)]
        pltpu.async_copy(h_hbm.at[ssel_v.at[pl.ds(0, B2)]], rows_v, sem).wait()

        def scale(g, _):
            eids = g * 16 + iota
            maskv = msel_v[pl.ds(g * 16, 16)]
            for f in range(32):
                fv = jnp.full((16,), f, jnp.int32)
                v = plsc.load_gather(rows_v, [eids, fv]) * maskv
                plsc.store_scatter(rows_v, [eids, fv], v)
            return 0
        lax.fori_loop(0, B2 // 16, scale, 0)
        pltpu.sync_copy(rows_v, acc_sh.at[dbat_v], add=True)
        for j in range((SEL - B2) // 16):
            ssel_v[pl.ds(j * 16, 16)] = ssel_v[pl.ds(B2 + j * 16, 16)]
            dsel_v[pl.ds(j * 16, 16)] = dsel_v[pl.ds(B2 + j * 16, 16)]
            msel_v[pl.ds(j * 16, 16)] = msel_v[pl.ds(B2 + j * 16, 16)]

    tile_base = sid * (E // NS)

    def chunk(ch, cnt_v):
        eb = pl.multiple_of(tile_base + ch * C2, 8)
        mb = pl.multiple_of((tile_base + ch * C2) // 2, 8)
        pltpu.sync_copy(src_hbm.at[pl.ds(eb, C2)], src_v)
        pltpu.sync_copy(dst_hbm.at[pl.ds(eb, C2)], dst_v)
        pltpu.sync_copy(sg_hbm.at[pl.ds(mb, C2 // 2)], sig_v)

        def group(g, cnt_v):
            eids = g * 16 + iota
            s16 = src_v[pl.ds(g * 16, 16)]
            d16 = dst_v[pl.ds(g * 16, 16)] - lo
            inr = jnp.logical_and(d16 >= 0, d16 < 50000)
            mv = plsc.load_gather(sig_v, [lax.shift_right_logical(eids, 1)])
            cum = plsc.cumsum(jnp.where(inr, 1, 0))
            pos = cnt_v + cum - 1
            plsc.store_scatter(ssel_v, [pos], s16, mask=inr)
            plsc.store_scatter(dsel_v, [pos], d16, mask=inr)
            plsc.store_scatter(msel_v, [pos], mv, mask=inr)
            return cnt_v + plsc.all_reduce_population_count(inr)
        cnt_v = lax.fori_loop(0, C2 // 16, group, cnt_v)

        for _rep in range(2):
            t = jnp.max(cnt_v)

            @pl.when(t >= B2)
            def _proc():
                process_batch()
            cnt_v = jnp.where(cnt_v >= B2, cnt_v - B2, cnt_v)
        return cnt_v

    cnt_v = lax.fori_loop(0, L2_CHUNKS, chunk, jnp.zeros((16,), jnp.int32))

    c_end = jnp.max(cnt_v)
    for j in range(B2 // 16):
        lm = (j * 16 + iota) >= c_end
        sl = pl.ds(j * 16, 16)
        msel_v[sl] = jnp.where(lm, 0.0, msel_v[sl])
        ssel_v[sl] = jnp.where(lm, 0, ssel_v[sl])
        dsel_v[sl] = jnp.where(lm, DUMP, dsel_v[sl])
    process_batch()
    plsc.subcore_barrier()

    r0 = sid * rows_per_tile
    for p in _pieces(rows_per_tile, B2):
        pltpu.sync_copy(acc_sh.at[pl.ds(r0, p)], rows_v.at[pl.ds(0, p)])
        pltpu.sync_copy(rows_v.at[pl.ds(0, p)], out_hbm.at[cid].at[pl.ds(r0, p)])
        r0 = r0 + p


def _tc_dense1_body(p_ref, w1a_ref, b1a_ref, w1b_ref, b1b_ref, h_ref):
    s = p_ref[0] + p_ref[1]
    deg = s[:, 15:16]
    hn = s * (1.0 / jnp.maximum(deg, 1.0))
    h1 = jnp.maximum(
        jnp.dot(hn, w1a_ref[...], preferred_element_type=jnp.float32)
        + b1a_ref[...], 0.0)
    h2 = jnp.maximum(
        jnp.dot(h1, w1b_ref[...], preferred_element_type=jnp.float32)
        + b1b_ref[...], 0.0)
    h_ref[...] = h2


def _tc_dense1(out1, w1a_pad, b1a, w1bT, b1b):
    B = 2048
    grid = (NPAD // B,)
    return pl.pallas_call(
        _tc_dense1_body,
        grid=grid,
        in_specs=[
            pl.BlockSpec((NC, B, 16), lambda i: (0, i, 0)),
            pl.BlockSpec((16, 32), lambda i: (0, 0)),
            pl.BlockSpec((1, 32), lambda i: (0, 0)),
            pl.BlockSpec((32, 32), lambda i: (0, 0)),
            pl.BlockSpec((1, 32), lambda i: (0, 0)),
        ],
        out_specs=pl.BlockSpec((B, 32), lambda i: (i, 0)),
        out_shape=jax.ShapeDtypeStruct((NPAD, 32), jnp.float32),
    )(out1, w1a_pad, b1a, w1bT, b1b)


def _tc_dense2_body(nsteps, a_ref, s1_ref, w2a_ref, b2a_ref, w2b_ref, b2b_ref,
                    wm1_ref, bm1_ref, wm2_ref, bm2_ref, out_ref, gmax):
    i = pl.program_id(0)

    @pl.when(i == 0)
    def _init():
        gmax[...] = jnp.full((1, 32), -jnp.inf, jnp.float32)

    a = a_ref[0]
    deg = s1_ref[0][:, 15:16]
    hn = a * (1.0 / jnp.maximum(deg, 1.0))
    h1 = jnp.maximum(
        jnp.dot(hn, w2a_ref[...], preferred_element_type=jnp.float32)
        + b2a_ref[...], 0.0)
    h2 = jnp.maximum(
        jnp.dot(h1, w2b_ref[...], preferred_element_type=jnp.float32)
        + b2b_ref[...], 0.0)
    gmax[...] = jnp.maximum(gmax[...], jnp.max(h2, axis=0, keepdims=True))

    @pl.when(i == nsteps - 1)
    def _head():
        g = gmax[...]
        z1 = jnp.maximum(
            jnp.dot(g, wm1_ref[...], preferred_element_type=jnp.float32)
            + bm1_ref[...], 0.0)
        logits = (jnp.dot(z1, wm2_ref[...], preferred_element_type=jnp.float32)
                  + bm2_ref[...])
        m = jnp.max(logits, axis=1, keepdims=True)
        e = jnp.exp(logits - m)
        out_ref[...] = e / jnp.sum(e, axis=1, keepdims=True)


def _tc_dense2(acc2, out1, w2aT, b2a, w2bT, b2b, wm1T, bm1, wm2T, bm2):
    B = 2000
    per_part = 50000 // B
    nsteps = NC * per_part
    return pl.pallas_call(
        functools.partial(_tc_dense2_body, nsteps),
        grid=(nsteps,),
        in_specs=[
            pl.BlockSpec((1, B, 32), lambda i: (i // 25, i % 25, 0)),
            pl.BlockSpec((1, B, 16), lambda i: (0, i, 0)),
            pl.BlockSpec((32, 32), lambda i: (0, 0)),
            pl.BlockSpec((1, 32), lambda i: (0, 0)),
            pl.BlockSpec((32, 32), lambda i: (0, 0)),
            pl.BlockSpec((1, 32), lambda i: (0, 0)),
            pl.BlockSpec((32, 16), lambda i: (0, 0)),
            pl.BlockSpec((1, 16), lambda i: (0, 0)),
            pl.BlockSpec((16, 2), lambda i: (0, 0)),
            pl.BlockSpec((1, 2), lambda i: (0, 0)),
        ],
        out_specs=pl.BlockSpec((1, 2), lambda i: (0, 0)),
        out_shape=jax.ShapeDtypeStruct((1, 2), jnp.float32),
        scratch_shapes=[pltpu.VMEM((1, 32), jnp.float32)],
    )(acc2, out1, w2aT, b2a, w2bT, b2b, wm1T, bm1, wm2T, bm2)


def kernel(x, edge_feat, edge_index, edge_mask,
           W1a, b1a, W1b, b1b, W2a, b2a, W2b, b2b, Wm1, bm1, Wm2, bm2):
    src, dst = _tc_split(edge_index.astype(jnp.int32))
    sg = _tc_sigmoid(edge_mask)

    x_pad = jnp.pad(x, ((0, NPAD - N), (4, 5)))
    w1a_pad = jnp.zeros((16, 32), jnp.float32).at[:11, :].set(W1a.T)

    out1 = _sc_layer1(src, dst, sg, edge_feat, x_pad)
    h = _tc_dense1(out1, w1a_pad, b1a.reshape(1, 32), W1b.T,
                   b1b.reshape(1, 32))
    acc2 = _sc_layer2(src, dst, sg, h)
    pred = _tc_dense2(acc2, out1, W2a.T, b2a.reshape(1, 32), W2b.T,
                      b2b.reshape(1, 32), Wm1.T, bm1.reshape(1, 16),
                      Wm2.T, bm2.reshape(1, 2))
    return pred


# speedup vs baseline: 1.0676x; 1.0669x over previous
import functools

import jax
import jax.numpy as jnp
from jax import lax
from jax.experimental import pallas as pl
from jax.experimental.pallas import tpu as pltpu
from jax.experimental.pallas import tpu_sc as plsc

N = 100000
E = 6400000
NPAD = 100352
NHALF = 50176
DUMP = 50008
NC = 2
NS = 16
C1 = 800
C2 = 640
B2 = 512
SEL = 1168
L1_CHUNKS = E // NC // NS // C1
L2_CHUNKS = E // NS // C2

def _pieces(total, c):
    out, r = [], 0
    while r < total:
        out.append(min(c, total - r)); r += out[-1]
    return out

_mesh = plsc.VectorSubcoreMesh(core_axis_name="c", subcore_axis_name="s",
                               num_cores=NC, num_subcores=NS)
_sc_params = pltpu.CompilerParams(needs_layout_passes=False,
                                  use_tc_tiling_on_sc=False)


def _tc_sigmoid_body(em_ref, sig_ref):
    m = em_ref[...]
    sig_ref[...] = (1.0 / (1.0 + jnp.exp(-m))).reshape(-1)


def _tc_sigmoid(edge_mask):
    B = 32768
    return pl.pallas_call(
        _tc_sigmoid_body,
        grid=(pl.cdiv(E // 2, B),),
        in_specs=[pl.BlockSpec((B, 1), lambda i: (i, 0))],
        out_specs=pl.BlockSpec((B,), lambda i: (i,)),
        out_shape=jax.ShapeDtypeStruct((E // 2,), jnp.float32),
    )(edge_mask)


def _tc_split_body(ei_ref, src_ref, dst_ref):
    e = ei_ref[...]
    src_ref[...] = e[0]
    dst_ref[...] = e[1]


def _tc_split(ei):
    B = 32768
    return pl.pallas_call(
        _tc_split_body,
        grid=(pl.cdiv(E, B),),
        in_specs=[pl.BlockSpec((2, B), lambda i: (0, i))],
        out_specs=[pl.BlockSpec((B,), lambda i: (i,)),
                   pl.BlockSpec((B,), lambda i: (i,))],
        out_shape=[jax.ShapeDtypeStruct((E,), jnp.int32),
                   jax.ShapeDtypeStruct((E,), jnp.int32)],
    )(ei)


def _tc_cols_body(ef_ref, o0, o1, o2, o3):
    e = ef_ref[...]
    o0[...] = e[:, 0]
    o1[...] = e[:, 1]
    o2[...] = e[:, 2]
    o3[...] = e[:, 3]


def _tc_cols(ef):
    B = 32768
    return pl.pallas_call(
        _tc_cols_body,
        grid=(pl.cdiv(E, B),),
        in_specs=[pl.BlockSpec((B, 4), lambda i: (i, 0))],
        out_specs=[pl.BlockSpec((B,), lambda i: (i,))] * 4,
        out_shape=[jax.ShapeDtypeStruct((E,), jnp.float32)] * 4,
    )(ef)


def _zero_rows(rows_v, nrows, ncols):
    z = jnp.zeros((16,), jnp.float32)
    def body(i, _):
        for c0 in range(0, ncols, 16):
            rows_v[i, pl.ds(c0, 16)] = z
        return 0
    lax.fori_loop(0, nrows, body, 0)


@functools.partial(
    pl.kernel,
    out_type=jax.ShapeDtypeStruct((NC, NPAD, 16), jnp.float32),
    mesh=_mesh,
    scratch_types=[
        pltpu.VMEM_SHARED((NPAD, 16), jnp.float32),
        pltpu.VMEM((C1,), jnp.int32),
        pltpu.VMEM((C1,), jnp.int32),
        pltpu.VMEM((C1 // 2,), jnp.float32),
        pltpu.VMEM((C1 * 4,), jnp.float32),
        pltpu.VMEM((C1, 16), jnp.float32),
        pltpu.SemaphoreType.DMA,
    ],
    compiler_params=_sc_params,
)
def _sc_layer1(src_hbm, dst_hbm, sg_hbm, ef0_hbm, ef1_hbm, ef2_hbm, ef3_hbm,
               xpad_hbm, out_hbm,
               acc_sh, src_v, dst_v, sig_v, ef_v, rows_v, sem):
    cid = lax.axis_index("c")
    sid = lax.axis_index("s")
    iota = lax.iota(jnp.int32, 16)
    ones = jnp.ones((16,), jnp.float32)
    f15 = jnp.full((16,), 15, jnp.int32)

    _zero_rows(rows_v, C1, 16)
    rows_per_tile = NPAD // NS
    r0 = sid * rows_per_tile
    for p in _pieces(rows_per_tile, C1):
        pltpu.sync_copy(rows_v.at[pl.ds(0, p)], acc_sh.at[pl.ds(r0, p)])
        r0 = r0 + p
    plsc.subcore_barrier()

    tile_base = cid * (E // NC) + sid * (E // NC // NS)

    def chunk(ch, _):
        eb = pl.multiple_of(tile_base + ch * C1, 8)
        mb = pl.multiple_of((tile_base + ch * C1) // 2, 8)
        pltpu.sync_copy(src_hbm.at[pl.ds(eb, C1)], src_v)
        pltpu.sync_copy(dst_hbm.at[pl.ds(eb, C1)], dst_v)
        pltpu.sync_copy(sg_hbm.at[pl.ds(mb, C1 // 2)], sig_v)
        for fi, efc in enumerate((ef0_hbm, ef1_hbm, ef2_hbm, ef3_hbm)):
            pltpu.sync_copy(efc.at[pl.ds(eb, C1)],
                            ef_v.at[pl.ds(fi * C1, C1)])
        pltpu.async_copy(xpad_hbm.at[src_v], rows_v, sem).wait()

        def group(g, _):
            eids = g * 16 + iota
            maskv = plsc.load_gather(sig_v, [lax.shift_right_logical(eids, 1)])
            for f in range(4):
                fv = jnp.full((16,), f, jnp.int32)
                v = plsc.load_gather(ef_v, [f * C1 + eids]) * maskv
                plsc.store_scatter(rows_v, [eids, fv], v)
            for f in range(4, 11):
                fv = jnp.full((16,), f, jnp.int32)
                v = plsc.load_gather(rows_v, [eids, fv]) * maskv
                plsc.store_scatter(rows_v, [eids, fv], v)
            plsc.store_scatter(rows_v, [eids, f15], ones)
            return 0
        lax.fori_loop(0, C1 // 16, group, 0)
        pltpu.sync_copy(rows_v, acc_sh.at[dst_v], add=True)
        return 0

    lax.fori_loop(0, L1_CHUNKS, chunk, 0)
    plsc.subcore_barrier()

    r0 = sid * rows_per_tile
    for p in _pieces(rows_per_tile, C1):
        pltpu.sync_copy(acc_sh.at[pl.ds(r0, p)], rows_v.at[pl.ds(0, p)])
        pltpu.sync_copy(rows_v.at[pl.ds(0, p)], out_hbm.at[cid].at[pl.ds(r0, p)])
        r0 = r0 + p


@functools.partial(
    pl.kernel,
    out_type=jax.ShapeDtypeStruct((NC, NHALF, 32), jnp.float32),
    mesh=_mesh,
    scratch_types=[
        pltpu.VMEM_SHARED((NHALF, 32), jnp.float32),
        pltpu.VMEM((C2,), jnp.int32),
        pltpu.VMEM((C2,), jnp.int32),
        pltpu.VMEM((C2 // 2,), jnp.float32),
        pltpu.VMEM((SEL,), jnp.int32),
        pltpu.VMEM((SEL,), jnp.int32),
        pltpu.VMEM((SEL,), jnp.float32),
        pltpu.VMEM((B2,), jnp.int32),
        pltpu.VMEM((B2, 32), jnp.float32),
        pltpu.SemaphoreType.DMA,
    ],
    compiler_params=_sc_params,
)
def _sc_layer2(src_hbm, dst_hbm, sg_hbm, h_hbm, out_hbm,
               acc_sh, src_v, dst_v, sig_v, ssel_v, dsel_v, msel_v,
               dbat_v, rows_v, sem):
    cid = lax.axis_index("c")
    sid = lax.axis_index("s")
    iota = lax.iota(jnp.int32, 16)
    lo = cid * 50000

    _zero_rows(rows_v, B2, 32)
    rows_per_tile = NHALF // NS
    r0 = sid * rows_per_tile
    for p in _pieces(rows_per_tile, B2):
        pltpu.sync_copy(rows_v.at[pl.ds(0, p)], acc_sh.at[pl.ds(r0, p)])
        r0 = r0 + p
    plsc.subcore_barrier()

    def process_batch():
        for j in range(B2 // 16):
            dbat_v[pl.ds(j * 16, 16)] = dsel_v[pl.ds(j * 16, 16)]
        pltpu.async_copy(h_hbm.at[ssel_v.at[pl.ds(0, B2)]], rows_v, sem).wait()

        def scale(g, _):
            eids = g * 16 + iota
            maskv = msel_v[pl.ds(g * 16, 16)]
            for f in range(32):
                fv = jnp.full((16,), f, jnp.int32)
                v = plsc.load_gather(rows_v, [eids, fv]) * maskv
                plsc.store_scatter(rows_v, [eids, fv], v)
            return 0
        lax.fori_loop(0, B2 // 16, scale, 0)
        pltpu.sync_copy(rows_v, acc_sh.at[dbat_v], add=True)
        for j in range((SEL - B2) // 16):
            ssel_v[pl.ds(j * 16, 16)] = ssel_v[pl.ds(B2 + j * 16, 16)]
            dsel_v[pl.ds(j * 16, 16)] = dsel_v[pl.ds(B2 + j * 16, 16)]
            msel_v[pl.ds(j * 16, 16)] = msel_v[pl.ds(B2 + j * 16, 16)]

    tile_base = sid * (E // NS)

    def chunk(ch, cnt_v):
        eb = pl.multiple_of(tile_base + ch * C2, 8)
        mb = pl.multiple_of((tile_base + ch * C2) // 2, 8)
        pltpu.sync_copy(src_hbm.at[pl.ds(eb, C2)], src_v)
        pltpu.sync_copy(dst_hbm.at[pl.ds(eb, C2)], dst_v)
        pltpu.sync_copy(sg_hbm.at[pl.ds(mb, C2 // 2)], sig_v)

        def group(g, cnt_v):
            eids = g * 16 + iota
            s16 = src_v[pl.ds(g * 16, 16)]
            d16 = dst_v[pl.ds(g * 16, 16)] - lo
            inr = jnp.logical_and(d16 >= 0, d16 < 50000)
            mv = plsc.load_gather(sig_v, [lax.shift_right_logical(eids, 1)])
            cum = plsc.cumsum(jnp.where(inr, 1, 0))
            pos = cnt_v + cum - 1
            plsc.store_scatter(ssel_v, [pos], s16, mask=inr)
            plsc.store_scatter(dsel_v, [pos], d16, mask=inr)
            plsc.store_scatter(msel_v, [pos], mv, mask=inr)
            return cnt_v + plsc.all_reduce_population_count(inr)
        cnt_v = lax.fori_loop(0, C2 // 16, group, cnt_v)

        for _rep in range(2):
            t = jnp.max(cnt_v)

            @pl.when(t >= B2)
            def _proc():
                process_batch()
            cnt_v = jnp.where(cnt_v >= B2, cnt_v - B2, cnt_v)
        return cnt_v

    cnt_v = lax.fori_loop(0, L2_CHUNKS, chunk, jnp.zeros((16,), jnp.int32))

    c_end = jnp.max(cnt_v)
    for j in range(B2 // 16):
        lm = (j * 16 + iota) >= c_end
        sl = pl.ds(j * 16, 16)
        msel_v[sl] = jnp.where(lm, 0.0, msel_v[sl])
        ssel_v[sl] = jnp.where(lm, 0, ssel_v[sl])
        dsel_v[sl] = jnp.where(lm, DUMP, dsel_v[sl])
    process_batch()
    plsc.subcore_barrier()

    r0 = sid * rows_per_tile
    for p in _pieces(rows_per_tile, B2):
        pltpu.sync_copy(acc_sh.at[pl.ds(r0, p)], rows_v.at[pl.ds(0, p)])
        pltpu.sync_copy(rows_v.at[pl.ds(0, p)], out_hbm.at[cid].at[pl.ds(r0, p)])
        r0 = r0 + p


def _tc_dense1_body(p_ref, w1a_ref, b1a_ref, w1b_ref, b1b_ref, h_ref):
    s = p_ref[0] + p_ref[1]
    deg = s[:, 15:16]
    hn = s * (1.0 / jnp.maximum(deg, 1.0))
    h1 = jnp.maximum(
        jnp.dot(hn, w1a_ref[...], preferred_element_type=jnp.float32)
        + b1a_ref[...], 0.0)
    h2 = jnp.maximum(
        jnp.dot(h1, w1b_ref[...], preferred_element_type=jnp.float32)
        + b1b_ref[...], 0.0)
    h_ref[...] = h2


def _tc_dense1(out1, w1a_pad, b1a, w1bT, b1b):
    B = 2048
    grid = (NPAD // B,)
    return pl.pallas_call(
        _tc_dense1_body,
        grid=grid,
        in_specs=[
            pl.BlockSpec((NC, B, 16), lambda i: (0, i, 0)),
            pl.BlockSpec((16, 32), lambda i: (0, 0)),
            pl.BlockSpec((1, 32), lambda i: (0, 0)),
            pl.BlockSpec((32, 32), lambda i: (0, 0)),
            pl.BlockSpec((1, 32), lambda i: (0, 0)),
        ],
        out_specs=pl.BlockSpec((B, 32), lambda i: (i, 0)),
        out_shape=jax.ShapeDtypeStruct((NPAD, 32), jnp.float32),
    )(out1, w1a_pad, b1a, w1bT, b1b)


def _tc_dense2_body(nsteps, a_ref, s1_ref, w2a_ref, b2a_ref, w2b_ref, b2b_ref,
                    wm1_ref, bm1_ref, wm2_ref, bm2_ref, out_ref, gmax):
    i = pl.program_id(0)

    @pl.when(i == 0)
    def _init():
        gmax[...] = jnp.full((1, 32), -jnp.inf, jnp.float32)

    a = a_ref[0]
    deg = s1_ref[0][:, 15:16]
    hn = a * (1.0 / jnp.maximum(deg, 1.0))
    h1 = jnp.maximum(
        jnp.dot(hn, w2a_ref[...], preferred_element_type=jnp.float32)
        + b2a_ref[...], 0.0)
    h2 = jnp.maximum(
        jnp.dot(h1, w2b_ref[...], preferred_element_type=jnp.float32)
        + b2b_ref[...], 0.0)
    gmax[...] = jnp.maximum(gmax[...], jnp.max(h2, axis=0, keepdims=True))

    @pl.when(i == nsteps - 1)
    def _head():
        g = gmax[...]
        z1 = jnp.maximum(
            jnp.dot(g, wm1_ref[...], preferred_element_type=jnp.float32)
            + bm1_ref[...], 0.0)
        logits = (jnp.dot(z1, wm2_ref[...], preferred_element_type=jnp.float32)
                  + bm2_ref[...])
        m = jnp.max(logits, axis=1, keepdims=True)
        e = jnp.exp(logits - m)
        out_ref[...] = e / jnp.sum(e, axis=1, keepdims=True)


def _tc_dense2(acc2, out1, w2aT, b2a, w2bT, b2b, wm1T, bm1, wm2T, bm2):
    B = 2000
    per_part = 50000 // B
    nsteps = NC * per_part
    return pl.pallas_call(
        functools.partial(_tc_dense2_body, nsteps),
        grid=(nsteps,),
        in_specs=[
            pl.BlockSpec((1, B, 32), lambda i: (i // 25, i % 25, 0)),
            pl.BlockSpec((1, B, 16), lambda i: (0, i, 0)),
            pl.BlockSpec((32, 32), lambda i: (0, 0)),
            pl.BlockSpec((1, 32), lambda i: (0, 0)),
            pl.BlockSpec((32, 32), lambda i: (0, 0)),
            pl.BlockSpec((1, 32), lambda i: (0, 0)),
            pl.BlockSpec((32, 16), lambda i: (0, 0)),
            pl.BlockSpec((1, 16), lambda i: (0, 0)),
            pl.BlockSpec((16, 2), lambda i: (0, 0)),
            pl.BlockSpec((1, 2), lambda i: (0, 0)),
        ],
        out_specs=pl.BlockSpec((1, 2), lambda i: (0, 0)),
        out_shape=jax.ShapeDtypeStruct((1, 2), jnp.float32),
        scratch_shapes=[pltpu.VMEM((1, 32), jnp.float32)],
    )(acc2, out1, w2aT, b2a, w2bT, b2b, wm1T, bm1, wm2T, bm2)


def kernel(x, edge_feat, edge_index, edge_mask,
           W1a, b1a, W1b, b1b, W2a, b2a, W2b, b2b, Wm1, bm1, Wm2, bm2):
    src, dst = _tc_split(edge_index.astype(jnp.int32))
    sg = _tc_sigmoid(edge_mask)
    ef0, ef1, ef2, ef3 = _tc_cols(edge_feat)

    x_pad = jnp.pad(x, ((0, NPAD - N), (4, 5)))
    w1a_pad = jnp.zeros((16, 32), jnp.float32).at[:11, :].set(W1a.T)

    out1 = _sc_layer1(src, dst, sg, ef0, ef1, ef2, ef3, x_pad)
    h = _tc_dense1(out1, w1a_pad, b1a.reshape(1, 32), W1b.T,
                   b1b.reshape(1, 32))
    acc2 = _sc_layer2(src, dst, sg, h)
    pred = _tc_dense2(acc2, out1, W2a.T, b2a.reshape(1, 32), W2b.T,
                      b2b.reshape(1, 32), Wm1.T, bm1.reshape(1, 16),
                      Wm2.T, bm2.reshape(1, 2))
    return pred
